# baseline (speedup 1.0000x reference)
import functools

import jax
import jax.numpy as jnp
from jax import lax
from jax.experimental import pallas as pl
from jax.experimental.pallas import tpu as pltpu
from jax.experimental.pallas import tpu_sc as plsc

USER_NUM_ = 1000000
ITEM_NUM_ = 100000
DIM_ = 64
BATCH_ = 16384

NC = 2
NS = 16
NW = NC * NS
B_PER_W = BATCH_ // NW
N_IDX_ROWS = B_PER_W // 128


def _gmf_body(users_hbm, items_hbm, ut_hbm, it_hbm, w_hbm, b_hbm, out_hbm,
              uidx, iidx, urows, irows, wv, bv, outv, sem):
    c = lax.axis_index("c")
    s = lax.axis_index("s")
    wid = s * NC + c

    pltpu.sync_copy(users_hbm.at[wid], uidx)
    pltpu.sync_copy(items_hbm.at[wid], iidx)
    pltpu.sync_copy(w_hbm, wv)
    pltpu.sync_copy(b_hbm, bv)

    cps = []
    for j in range(N_IDX_ROWS):
        cps.append(pltpu.async_copy(ut_hbm.at[uidx.at[j]],
                                    urows.at[pl.ds(j * 128, 128)], sem))
        cps.append(pltpu.async_copy(it_hbm.at[iidx.at[j]],
                                    irows.at[pl.ds(j * 128, 128)], sem))
    for cp in cps:
        cp.wait()

    lane = lax.iota(jnp.int32, 16)
    bias = bv[:]

    def gbody(g, _):
        rows = g * 16 + lane
        wchunks = [wv[pl.ds(k * 16, 16)] for k in range(DIM_ // 16)]
        accs = [bias,
                jnp.zeros((16,), jnp.float32),
                jnp.zeros((16,), jnp.float32),
                jnp.zeros((16,), jnp.float32)]
        for d in range(DIM_):
            col = jnp.full((16,), d, jnp.int32)
            uu = plsc.load_gather(urows, [rows, col])
            vv = plsc.load_gather(irows, [rows, col])
            wd = wchunks[d // 16][d % 16]
            accs[d % 4] = accs[d % 4] + uu * vv * wd
        acc = (accs[0] + accs[1]) + (accs[2] + accs[3])
        outv[pl.ds(g * 16, 16)] = acc
        return 0

    lax.fori_loop(0, B_PER_W // 16, gbody, 0)

    pltpu.sync_copy(outv, out_hbm.at[pl.ds(wid * B_PER_W, B_PER_W)])


@jax.jit
def _gmf_call(users_r, items_r, user_table, item_table, w_flat, bias_vec):
    mesh = plsc.VectorSubcoreMesh(core_axis_name="c", subcore_axis_name="s")
    return pl.kernel(
        _gmf_body,
        mesh=mesh,
        compiler_params=pltpu.CompilerParams(
            needs_layout_passes=False, use_tc_tiling_on_sc=False),
        out_type=jax.ShapeDtypeStruct((BATCH_,), jnp.float32),
        scratch_types=[
            pltpu.VMEM((N_IDX_ROWS, 128), jnp.int32),
            pltpu.VMEM((N_IDX_ROWS, 128), jnp.int32),
            pltpu.VMEM((B_PER_W, DIM_), jnp.float32),
            pltpu.VMEM((B_PER_W, DIM_), jnp.float32),
            pltpu.VMEM((DIM_,), jnp.float32),
            pltpu.VMEM((16,), jnp.float32),
            pltpu.VMEM((B_PER_W,), jnp.float32),
            pltpu.SemaphoreType.DMA,
        ],
    )(users_r, items_r, user_table, item_table, w_flat, bias_vec)


def kernel(users, items, user_table, item_table, beta_w, beta_b):
    users_r = users.astype(jnp.int32).reshape(NW, N_IDX_ROWS, 128)
    items_r = items.astype(jnp.int32).reshape(NW, N_IDX_ROWS, 128)
    w_flat = beta_w.reshape(DIM_)
    bias_vec = jnp.broadcast_to(beta_b, (16,))
    out = _gmf_call(users_r, items_r, user_table, item_table, w_flat, bias_vec)
    return out.reshape(BATCH_, 1)


# speedup vs baseline: 1.5993x; 1.5993x over previous
import functools

import jax
import jax.numpy as jnp
from jax import lax
from jax.experimental import pallas as pl
from jax.experimental.pallas import tpu as pltpu
from jax.experimental.pallas import tpu_sc as plsc

USER_NUM_ = 1000000
ITEM_NUM_ = 100000
DIM_ = 64
BATCH_ = 16384

NC = 2
NS = 16
NW = NC * NS
B_PER_W = BATCH_ // NW
N_GROUPS = B_PER_W // 16


def _gmf_body(users_hbm, items_hbm, ut_hbm, it_hbm, w_hbm, b_hbm, out_hbm,
              uidx, iidx, urows, irows, wv, bv, outv, u_s, i_s, sem):
    c = lax.axis_index("c")
    s = lax.axis_index("s")
    wid = s * NC + c
    base = wid * B_PER_W

    pltpu.sync_copy(users_hbm.at[pl.ds(base, B_PER_W)], uidx)
    pltpu.sync_copy(items_hbm.at[pl.ds(base, B_PER_W)], iidx)
    pltpu.sync_copy(w_hbm, wv)
    pltpu.sync_copy(b_hbm, bv)

    def mirror(g, _):
        uvec = uidx[pl.ds(g * 16, 16)]
        ivec = iidx[pl.ds(g * 16, 16)]
        for i in range(16):
            u_s[g * 16 + i] = uvec[i]
            i_s[g * 16 + i] = ivec[i]
        return 0

    lax.fori_loop(0, N_GROUPS, mirror, 0)

    lane = lax.iota(jnp.int32, 16)
    bias = bv[:]
    HALF = B_PER_W // 2

    for p in range(2):
        off = p * HALF

        def fire(q, _):
            for i in range(4):
                j = q * 4 + i
                pltpu.async_copy(ut_hbm.at[pl.ds(u_s[off + j], 1)],
                                 urows.at[pl.ds(j, 1)], sem)
                pltpu.async_copy(it_hbm.at[pl.ds(i_s[off + j], 1)],
                                 irows.at[pl.ds(j, 1)], sem)
            return 0

        lax.fori_loop(0, HALF // 4, fire, 0)

        pltpu.make_async_copy(ut_hbm.at[pl.ds(0, HALF)], urows, sem).wait()
        pltpu.make_async_copy(it_hbm.at[pl.ds(0, HALF)], irows, sem).wait()

        def gbody(g, _):
            rows = g * 16 + lane
            wchunks = [wv[pl.ds(k * 16, 16)] for k in range(DIM_ // 16)]
            accs = [bias,
                    jnp.zeros((16,), jnp.float32),
                    jnp.zeros((16,), jnp.float32),
                    jnp.zeros((16,), jnp.float32)]
            for d in range(DIM_):
                col = jnp.full((16,), d, jnp.int32)
                uu = plsc.load_gather(urows, [rows, col])
                vv = plsc.load_gather(irows, [rows, col])
                wd = wchunks[d // 16][d % 16]
                accs[d % 4] = accs[d % 4] + uu * vv * wd
            acc = (accs[0] + accs[1]) + (accs[2] + accs[3])
            outv[pl.ds(off + g * 16, 16)] = acc
            return 0

        lax.fori_loop(0, HALF // 16, gbody, 0)

    pltpu.sync_copy(outv, out_hbm.at[pl.ds(base, B_PER_W)])


@jax.jit
def _gmf_call(users, items, user_table, item_table, w_flat, bias_vec):
    mesh = plsc.VectorSubcoreMesh(core_axis_name="c", subcore_axis_name="s")
    return pl.kernel(
        _gmf_body,
        mesh=mesh,
        compiler_params=pltpu.CompilerParams(needs_layout_passes=False),
        out_type=jax.ShapeDtypeStruct((BATCH_,), jnp.float32),
        scratch_types=[
            pltpu.VMEM((B_PER_W,), jnp.int32),
            pltpu.VMEM((B_PER_W,), jnp.int32),
            pltpu.VMEM((B_PER_W // 2, DIM_), jnp.float32),
            pltpu.VMEM((B_PER_W // 2, DIM_), jnp.float32),
            pltpu.VMEM((DIM_,), jnp.float32),
            pltpu.VMEM((16,), jnp.float32),
            pltpu.VMEM((B_PER_W,), jnp.float32),
            pltpu.SMEM((B_PER_W,), jnp.int32),
            pltpu.SMEM((B_PER_W,), jnp.int32),
            pltpu.SemaphoreType.DMA,
        ],
    )(users, items, user_table, item_table, w_flat, bias_vec)


def kernel(users, items, user_table, item_table, beta_w, beta_b):
    users_i = users.astype(jnp.int32)
    items_i = items.astype(jnp.int32)
    w_flat = beta_w.reshape(DIM_)
    bias_vec = jnp.broadcast_to(beta_b, (16,))
    out = _gmf_call(users_i, items_i, user_table, item_table, w_flat, bias_vec)
    return out.reshape(BATCH_, 1)
